# baseline (speedup 1.0000x reference)
import functools
import math

import jax
import jax.numpy as jnp
import numpy as np
from jax import lax
from jax.experimental import pallas as pl
from jax.experimental.pallas import tpu as pltpu
from jax.experimental.pallas import tpu_sc as plsc

_VOCAB = 100000
_D = 768
_B = 4
_S = 4096
_NC = 2
_NS = 16
_NW = _NC * _NS
_POS_PER_W = _S // _NW
_P = 8
_NUNITS = _POS_PER_W // _P
_ROWS = _B * _P
_LANES = 16
_NVEC = _D // _LANES
_JGRP = 4
_SCALE = math.sqrt(_D)


def _make_pe_np(max_len, d_model):
    pe = np.zeros((max_len, d_model), dtype=np.float32)
    position = np.arange(0, max_len, dtype=np.float32)[:, None]
    div_term = np.exp(
        np.arange(0, d_model, 2, dtype=np.float32) * -(math.log(10000.0) / d_model)
    )
    pe[:, 0::2] = np.sin(position * div_term)
    pe[:, 1::2] = np.cos(position * div_term)
    return pe


_PE = _make_pe_np(_S, _D)


def _body(x_hbm, table_hbm, pe_hbm, out_hbm, idx_v, pe_v, gbuf, obuf,
          gsem, ssem, psem):
    wid = lax.axis_index("s") * _NC + lax.axis_index("c")
    pos0 = wid * _POS_PER_W

    pltpu.sync_copy(x_hbm.at[wid], idx_v)

    def gather(u):
        return pltpu.async_copy(table_hbm.at[idx_v.at[u]], gbuf.at[u % 2], gsem)

    def pe_load(u):
        return pltpu.async_copy(
            pe_hbm.at[pl.ds(pos0 + u * _P, _P)], pe_v.at[u % 2], psem)

    def scatters(u):
        return [
            pltpu.async_copy(
                obuf.at[u % 2, pl.ds(b * _P, _P)],
                out_hbm.at[pl.ds(b * _S + pos0 + u * _P, _P)],
                ssem,
            )
            for b in range(_B)
        ]

    g = {0: gather(0), 1: gather(1)}
    p = {0: pe_load(0), 1: pe_load(1)}
    s = {}
    for u in range(_NUNITS):
        slot = u % 2
        g[u].wait()
        p[u].wait()
        if u >= 2:
            for cp in s[u - 2]:
                cp.wait()

        def pos_body(i, _, _slot=slot):
            def j_body(jg, _):
                for dj in range(_JGRP):
                    sl = pl.ds((jg * _JGRP + dj) * _LANES, _LANES)
                    pe_vec = pe_v[_slot, i, sl]
                    for b in range(_B):
                        r = b * _P + i
                        obuf[_slot, r, sl] = gbuf[_slot, r, sl] * _SCALE + pe_vec
                return 0

            lax.fori_loop(0, _NVEC // _JGRP, j_body, 0)
            return 0

        lax.fori_loop(0, _P, pos_body, 0)
        s[u] = scatters(u)
        if u + 2 < _NUNITS:
            g[u + 2] = gather(u + 2)
            p[u + 2] = pe_load(u + 2)
    for u in (_NUNITS - 2, _NUNITS - 1):
        for cp in s[u]:
            cp.wait()


def _build(interpret=False):
    mesh = plsc.VectorSubcoreMesh(core_axis_name="c", subcore_axis_name="s")
    return pl.kernel(
        _body,
        out_type=jax.ShapeDtypeStruct((_B * _S, _D), jnp.float32),
        mesh=mesh,
        scratch_types=[
            pltpu.VMEM((_NUNITS, _ROWS), jnp.int32),
            pltpu.VMEM((2, _P, _D), jnp.float32),
            pltpu.VMEM((2, _ROWS, _D), jnp.float32),
            pltpu.VMEM((2, _ROWS, _D), jnp.float32),
            pltpu.SemaphoreType.DMA,
            pltpu.SemaphoreType.DMA,
            pltpu.SemaphoreType.DMA,
        ],
        interpret=interpret,
    )


_sc_embed = _build()


def kernel(x, table):
    x2 = (
        x.astype(jnp.int32)
        .reshape(_B, _NW, _NUNITS, _P)
        .transpose(1, 2, 0, 3)
        .reshape(_NW, _NUNITS, _ROWS)
    )
    out = _sc_embed(x2, table, jnp.asarray(_PE))
    return out.reshape(_B, _S, _D)


# speedup vs baseline: 2.3785x; 2.3785x over previous
import functools
import math

import jax
import jax.numpy as jnp
import numpy as np
from jax import lax
from jax.experimental import pallas as pl
from jax.experimental.pallas import tpu as pltpu
from jax.experimental.pallas import tpu_sc as plsc

_VOCAB = 100000
_D = 768
_B = 4
_S = 4096
_NC = 2
_NS = 16
_NW = _NC * _NS
_POS_PER_W = _S // _NW
_P = 8
_NUNITS = _POS_PER_W // _P
_ROWS = _B * _P
_LANES = 16
_NVEC = _D // _LANES
_JGRP = 4
_SCALE = math.sqrt(_D)


def _make_pe_np(max_len, d_model):
    pe = np.zeros((max_len, d_model), dtype=np.float32)
    position = np.arange(0, max_len, dtype=np.float32)[:, None]
    div_term = np.exp(
        np.arange(0, d_model, 2, dtype=np.float32) * -(math.log(10000.0) / d_model)
    )
    pe[:, 0::2] = np.sin(position * div_term)
    pe[:, 1::2] = np.cos(position * div_term)
    return pe


_PE = _make_pe_np(_S, _D)


def _body(x_hbm, table_hbm, pe_hbm, out_hbm, idx_v, pe_v, gbuf, obuf,
          gsem, ssem, psem):
    wid = lax.axis_index("s") * _NC + lax.axis_index("c")
    pos0 = wid * _POS_PER_W

    pltpu.sync_copy(x_hbm.at[wid], idx_v)

    def gather(u):
        return pltpu.async_copy(table_hbm.at[idx_v.at[u]], gbuf.at[u % 2], gsem)

    def pe_load(u):
        return pltpu.async_copy(
            pe_hbm.at[pl.ds(pos0 + u * _P, _P)], pe_v.at[u % 2], psem)

    def scatters(u):
        return [
            pltpu.async_copy(
                obuf.at[u % 2, pl.ds(b * _P, _P)],
                out_hbm.at[pl.ds(b * _S + pos0 + u * _P, _P)],
                ssem,
            )
            for b in range(_B)
        ]

    g = {0: gather(0), 1: gather(1)}
    p = {0: pe_load(0), 1: pe_load(1)}
    s = {}
    for u in range(_NUNITS):
        slot = u % 2
        g[u].wait()
        p[u].wait()
        if u >= 2:
            for cp in s[u - 2]:
                cp.wait()

        def j_body(j, _, _slot=slot):
            sl = pl.ds(j * _LANES, _LANES)
            for i in range(_P):
                pe_vec = pe_v[_slot, i, sl]
                for b in range(_B):
                    r = b * _P + i
                    obuf[_slot, r, sl] = gbuf[_slot, r, sl] * _SCALE + pe_vec
            return 0

        lax.fori_loop(0, _NVEC, j_body, 0)
        s[u] = scatters(u)
        if u + 2 < _NUNITS:
            g[u + 2] = gather(u + 2)
            p[u + 2] = pe_load(u + 2)
    for u in (_NUNITS - 2, _NUNITS - 1):
        for cp in s[u]:
            cp.wait()


def _build(interpret=False):
    mesh = plsc.VectorSubcoreMesh(core_axis_name="c", subcore_axis_name="s")
    return pl.kernel(
        _body,
        out_type=jax.ShapeDtypeStruct((_B * _S, _D), jnp.float32),
        mesh=mesh,
        scratch_types=[
            pltpu.VMEM((_NUNITS, _ROWS), jnp.int32),
            pltpu.VMEM((2, _P, _D), jnp.float32),
            pltpu.VMEM((2, _ROWS, _D), jnp.float32),
            pltpu.VMEM((2, _ROWS, _D), jnp.float32),
            pltpu.SemaphoreType.DMA,
            pltpu.SemaphoreType.DMA,
            pltpu.SemaphoreType.DMA,
        ],
        interpret=interpret,
    )


_sc_embed = _build()


def kernel(x, table):
    x2 = (
        x.astype(jnp.int32)
        .reshape(_B, _NW, _NUNITS, _P)
        .transpose(1, 2, 0, 3)
        .reshape(_NW, _NUNITS, _ROWS)
    )
    out = _sc_embed(x2, table, jnp.asarray(_PE))
    return out.reshape(_B, _S, _D)
